# baseline (speedup 1.0000x reference)
import functools

import jax
import jax.numpy as jnp
from jax import lax
from jax.experimental import pallas as pl
from jax.experimental.pallas import tpu as pltpu
from jax.experimental.pallas import tpu_sc as plsc

_N_EMB = 1024
_DIM = 64
_ROWS = 16 * 1024
_BLK = 1024
_GRID = _ROWS // _BLK

_NC = 2
_NS = 16
_NW = _NC * _NS
_BPW = _ROWS // _NW


_BPS = 4


def _dist_argmin_body(xt_ref, cbt_ref, idx_ref, part_ref):
    cbt = cbt_ref[...]
    colsum = jnp.sum(cbt * cbt, axis=0)
    for b in range(_BPS):
        xt = xt_ref[b]
        rowsum = jnp.sum(xt * xt, axis=0)
        mmt = lax.dot_general(
            cbt, xt, (((0,), (0,)), ((), ())),
            preferred_element_type=jnp.float32,
        )
        dist = colsum[:, None] + rowsum[None, :] - 2.0 * mmt
        minval = jnp.min(dist, axis=0)
        iota = lax.broadcasted_iota(jnp.int32, dist.shape, 0)
        idx = jnp.min(
            jnp.where(dist == minval[None, :], iota, jnp.int32(_N_EMB)),
            axis=0,
        )
        idx_ref[b, 0, :] = idx
        part_ref[b, 0, :] = jnp.full((128,), jnp.sum(minval), jnp.float32)


def _tc_stage(xt, cbt):
    nb = xt.shape[0]
    idx3, part3 = pl.pallas_call(
        _dist_argmin_body,
        grid=(nb // _BPS,),
        in_specs=[
            pl.BlockSpec((_BPS, _DIM, _BLK), lambda i: (i, 0, 0)),
            pl.BlockSpec((_DIM, _N_EMB), lambda i: (0, 0)),
        ],
        out_specs=[
            pl.BlockSpec((_BPS, 1, _BLK), lambda i: (i, 0, 0)),
            pl.BlockSpec((_BPS, 1, 128), lambda i: (i, 0, 0)),
        ],
        out_shape=[
            jax.ShapeDtypeStruct((nb, 1, _BLK), jnp.int32),
            jax.ShapeDtypeStruct((nb, 1, 128), jnp.float32),
        ],
    )(xt, cbt)
    return idx3.reshape(nb * _BLK), part3[:, 0, 0]


def _sc_gather(cb128, idx):
    bpw = _ROWS // _NW
    mesh = plsc.VectorSubcoreMesh(core_axis_name="c", subcore_axis_name="s")

    @functools.partial(
        pl.kernel,
        mesh=mesh,
        compiler_params=pltpu.CompilerParams(use_tc_tiling_on_sc=True),
        out_type=jax.ShapeDtypeStruct((_ROWS, 128), jnp.float32),
        scratch_types=[
            pltpu.VMEM((bpw,), jnp.int32),
            pltpu.VMEM((bpw, 128), jnp.float32),
            pltpu.SemaphoreType.DMA,
        ],
    )
    def k(cb_hbm, idx_hbm, out_hbm, idx_v, rows_v, sem):
        wid = lax.axis_index("s") * _NC + lax.axis_index("c")
        base = wid * bpw
        pltpu.sync_copy(idx_hbm.at[pl.ds(base, bpw)], idx_v)
        pltpu.async_copy(cb_hbm.at[idx_v], rows_v, sem).wait()
        pltpu.sync_copy(rows_v, out_hbm.at[pl.ds(base, bpw)])

    return k(cb128, idx)


def _transpose_body(q_ref, out_ref):
    for b in range(4):
        out_ref[b] = q_ref[b][:, : _DIM].T


def _tc_transpose(q3):
    return pl.pallas_call(
        _transpose_body,
        grid=(4,),
        in_specs=[pl.BlockSpec((4, _BLK, 128), lambda i: (i, 0, 0))],
        out_specs=pl.BlockSpec((4, _DIM, _BLK), lambda i: (i, 0, 0)),
        out_shape=jax.ShapeDtypeStruct((16, _DIM, _BLK), jnp.float32),
    )(q3)


def kernel(x, codebook):
    xt = jnp.transpose(x, (0, 2, 1))
    cbt = codebook.T
    idx, part = _tc_stage(xt, cbt)
    cb128 = jnp.pad(codebook, ((0, 0), (0, 128 - _DIM)))
    q = _sc_gather(cb128, idx)
    qt = _tc_transpose(q.reshape(16, _BLK, 128))
    loss = 1.25 * (jnp.sum(part) / jnp.float32(_ROWS * _DIM))
    return jnp.transpose(qt, (0, 2, 1)), loss


# speedup vs baseline: 1.0141x; 1.0141x over previous
import functools

import jax
import jax.numpy as jnp
from jax import lax
from jax.experimental import pallas as pl
from jax.experimental.pallas import tpu as pltpu
from jax.experimental.pallas import tpu_sc as plsc

_N_EMB = 1024
_DIM = 64
_ROWS = 16 * 1024
_BLK = 1024
_GRID = _ROWS // _BLK

_NC = 2
_NS = 16
_NW = _NC * _NS
_BPW = _ROWS // _NW


_BPS = 4


def _dist_argmin_body(xt_ref, cbt_ref, idx_ref, part_ref):
    cbt = cbt_ref[...]
    colsum = jnp.sum(cbt * cbt, axis=0)
    for b in range(_BPS):
        xt = xt_ref[b]
        rowsum = jnp.sum(xt * xt, axis=0)
        mmt = lax.dot_general(
            cbt, xt, (((0,), (0,)), ((), ())),
            preferred_element_type=jnp.float32,
        )
        dist = colsum[:, None] + rowsum[None, :] - 2.0 * mmt
        minval = jnp.min(dist, axis=0)
        iota = lax.broadcasted_iota(jnp.int32, dist.shape, 0)
        idx = jnp.min(
            jnp.where(dist == minval[None, :], iota, jnp.int32(_N_EMB)),
            axis=0,
        )
        idx_ref[b, 0, :] = idx
        part_ref[b, 0, :] = jnp.full((128,), jnp.sum(minval), jnp.float32)


def _tc_stage(xt, cbt):
    nb = xt.shape[0]
    idx3, part3 = pl.pallas_call(
        _dist_argmin_body,
        grid=(nb // _BPS,),
        in_specs=[
            pl.BlockSpec((_BPS, _DIM, _BLK), lambda i: (i, 0, 0)),
            pl.BlockSpec((_DIM, _N_EMB), lambda i: (0, 0)),
        ],
        out_specs=[
            pl.BlockSpec((_BPS, 1, _BLK), lambda i: (i, 0, 0)),
            pl.BlockSpec((_BPS, 1, 128), lambda i: (i, 0, 0)),
        ],
        out_shape=[
            jax.ShapeDtypeStruct((nb, 1, _BLK), jnp.int32),
            jax.ShapeDtypeStruct((nb, 1, 128), jnp.float32),
        ],
    )(xt, cbt)
    return idx3.reshape(nb * _BLK), part3[:, 0, 0]


def _sc_gather(cb128, idx):
    bpw = _ROWS // _NW
    mesh = plsc.VectorSubcoreMesh(core_axis_name="c", subcore_axis_name="s")

    @functools.partial(
        pl.kernel,
        mesh=mesh,
        compiler_params=pltpu.CompilerParams(use_tc_tiling_on_sc=True),
        out_type=jax.ShapeDtypeStruct((_ROWS, 128), jnp.float32),
        scratch_types=[
            pltpu.VMEM((bpw,), jnp.int32),
            pltpu.VMEM((bpw, 128), jnp.float32),
            pltpu.SemaphoreType.DMA,
        ],
    )
    def k(cb_hbm, idx_hbm, out_hbm, idx_v, rows_v, sem):
        wid = lax.axis_index("s") * _NC + lax.axis_index("c")
        base = wid * bpw
        pltpu.sync_copy(idx_hbm.at[pl.ds(base, bpw)], idx_v)
        pltpu.async_copy(cb_hbm.at[idx_v], rows_v, sem).wait()
        pltpu.sync_copy(rows_v, out_hbm.at[pl.ds(base, bpw)])

    return k(cb128, idx)


def _transpose_body(q_ref, out_ref):
    for b in range(8):
        out_ref[b] = q_ref[b][:, : _DIM].T


def _tc_transpose(q3):
    return pl.pallas_call(
        _transpose_body,
        grid=(2,),
        in_specs=[pl.BlockSpec((8, _BLK, 128), lambda i: (i, 0, 0))],
        out_specs=pl.BlockSpec((8, _DIM, _BLK), lambda i: (i, 0, 0)),
        out_shape=jax.ShapeDtypeStruct((16, _DIM, _BLK), jnp.float32),
    )(q3)


def kernel(x, codebook):
    xt = jnp.transpose(x, (0, 2, 1))
    cbt = codebook.T
    idx, part = _tc_stage(xt, cbt)
    cb128 = jnp.pad(codebook, ((0, 0), (0, 128 - _DIM)))
    q = _sc_gather(cb128, idx)
    qt = _tc_transpose(q.reshape(16, _BLK, 128))
    loss = 1.25 * (jnp.sum(part) / jnp.float32(_ROWS * _DIM))
    return jnp.transpose(qt, (0, 2, 1)), loss


# speedup vs baseline: 1.0210x; 1.0068x over previous
import functools

import jax
import jax.numpy as jnp
from jax import lax
from jax.experimental import pallas as pl
from jax.experimental.pallas import tpu as pltpu
from jax.experimental.pallas import tpu_sc as plsc

_N_EMB = 1024
_DIM = 64
_ROWS = 16 * 1024
_BLK = 1024
_BPS = 4

_NC = 2
_NW = 32


def _dist_argmin_body(xt_ref, cbt_ref, idx_ref, part_ref):
    cbt = cbt_ref[...]
    colsum = jnp.sum(cbt * cbt, axis=0)
    for b in range(_BPS):
        xt = xt_ref[b]
        rowsum = jnp.sum(xt * xt, axis=0)
        mmt = lax.dot_general(
            cbt, xt, (((0,), (0,)), ((), ())),
            preferred_element_type=jnp.float32,
        )
        dist = colsum[:, None] + rowsum[None, :] - 2.0 * mmt
        minval = jnp.min(dist, axis=0)
        iota = lax.broadcasted_iota(jnp.int32, dist.shape, 0)
        idx = jnp.min(
            jnp.where(dist == minval[None, :], iota, jnp.int32(_N_EMB)),
            axis=0,
        )
        idx_ref[b, 0, :] = idx
        part_ref[b, 0, :] = jnp.full((128,), jnp.sum(minval), jnp.float32)


def _tc_stage(xt, cbt):
    nb = xt.shape[0]
    idx3, part3 = pl.pallas_call(
        _dist_argmin_body,
        grid=(nb // _BPS,),
        in_specs=[
            pl.BlockSpec((_BPS, _DIM, _BLK), lambda i: (i, 0, 0)),
            pl.BlockSpec((_DIM, _N_EMB), lambda i: (0, 0)),
        ],
        out_specs=[
            pl.BlockSpec((_BPS, 1, _BLK), lambda i: (i, 0, 0)),
            pl.BlockSpec((_BPS, 1, 128), lambda i: (i, 0, 0)),
        ],
        out_shape=[
            jax.ShapeDtypeStruct((nb, 1, _BLK), jnp.int32),
            jax.ShapeDtypeStruct((nb, 1, 128), jnp.float32),
        ],
    )(xt, cbt)
    return idx3.reshape(nb * _BLK), part3[:, 0, 0]


def _sc_gather(cb128, idx):
    bpw = _ROWS // _NW
    mesh = plsc.VectorSubcoreMesh(core_axis_name="c", subcore_axis_name="s")

    @functools.partial(
        pl.kernel,
        mesh=mesh,
        compiler_params=pltpu.CompilerParams(use_tc_tiling_on_sc=True),
        out_type=jax.ShapeDtypeStruct((_ROWS, 128), jnp.float32),
        scratch_types=[
            pltpu.VMEM((bpw,), jnp.int32),
            pltpu.VMEM((bpw, 128), jnp.float32),
            pltpu.SemaphoreType.DMA,
        ],
    )
    def k(cb_hbm, idx_hbm, out_hbm, idx_v, rows_v, sem):
        wid = lax.axis_index("s") * _NC + lax.axis_index("c")
        base = wid * bpw
        pltpu.sync_copy(idx_hbm.at[pl.ds(base, bpw)], idx_v)
        pltpu.async_copy(cb_hbm.at[idx_v], rows_v, sem).wait()
        pltpu.sync_copy(rows_v, out_hbm.at[pl.ds(base, bpw)])

    return k(cb128, idx)


def _transpose_body(q_ref, out_ref):
    for b in range(8):
        out_ref[b] = q_ref[b][:, : _DIM].T


def _tc_transpose(q3):
    return pl.pallas_call(
        _transpose_body,
        grid=(2,),
        in_specs=[pl.BlockSpec((8, _BLK, 128), lambda i: (i, 0, 0))],
        out_specs=pl.BlockSpec((8, _DIM, _BLK), lambda i: (i, 0, 0)),
        out_shape=jax.ShapeDtypeStruct((16, _DIM, _BLK), jnp.float32),
    )(q3)


def kernel(x, codebook):
    xt = jnp.transpose(x, (0, 2, 1))
    cbt = codebook.T
    idx, part = _tc_stage(xt, cbt)
    cb128 = jnp.pad(codebook, ((0, 0), (0, 128 - _DIM)))
    q = _sc_gather(cb128, idx)
    qt = _tc_transpose(q.reshape(16, _BLK, 128))
    loss = 1.25 * (jnp.sum(part) / jnp.float32(_ROWS * _DIM))
    return jnp.transpose(qt, (0, 2, 1)), loss
